# Initial kernel scaffold; baseline (speedup 1.0000x reference)
import jax
import jax.numpy as jnp
from jax.experimental import pallas as pl


def kernel(x, manyshotTemp, mediumshotTemp, fewshotTemp, manyshotBias, mediumshotBias, fewshotBias, many_mask, med_mask, few_mask):
    raise NotImplementedError("write your pallas kernel here")


# speedup vs baseline: 4.5779x; 4.5779x over previous
import functools
import jax
import jax.numpy as jnp
from jax.experimental import pallas as pl
from jax.experimental.pallas import tpu as pltpu

_B = 16384
_N = 1003
_SEG1 = 392
_SEG2 = 866
_BLK = 512


def _tc_body(x_ref, t_ref, b_ref, o_ref):
    z = x_ref[...] * t_ref[...] + b_ref[...]
    z1 = z[:, 0:_SEG1]
    z2 = z[:, _SEG1:_SEG2]
    z3 = z[:, _SEG2:_N]
    m1 = jnp.max(z1, axis=1, keepdims=True)
    m2 = jnp.max(z2, axis=1, keepdims=True)
    m3 = jnp.max(z3, axis=1, keepdims=True)
    e1 = jnp.exp(z1 - m1)
    e2 = jnp.exp(z2 - m2)
    e3 = jnp.exp(z3 - m3)
    s1 = jnp.sum(e1, axis=1, keepdims=True)
    s2 = jnp.sum(e2, axis=1, keepdims=True)
    s3 = jnp.sum(e3, axis=1, keepdims=True)
    renorm = 3.0 - e1[:, -1:] / s1 - e2[:, -1:] / s2 - e3[:, -1:] / s3
    lr = jnp.log(renorm)
    c1 = m1 + jnp.log(s1) + lr
    c2 = m2 + jnp.log(s2) + lr
    c3 = m3 + jnp.log(s3) + lr
    o_ref[:, 0:391] = z1[:, :-1] - c1
    o_ref[:, 391:864] = z2[:, :-1] - c2
    o_ref[:, 864:1000] = z3[:, :-1] - c3


@jax.jit
def _run(x, t, b):
    grid = _B // _BLK
    return pl.pallas_call(
        _tc_body,
        grid=(grid,),
        in_specs=[
            pl.BlockSpec((_BLK, _N), lambda i: (i, 0)),
            pl.BlockSpec((1, _N), lambda i: (0, 0)),
            pl.BlockSpec((1, _N), lambda i: (0, 0)),
        ],
        out_specs=pl.BlockSpec((_BLK, 1000), lambda i: (i, 0)),
        out_shape=jax.ShapeDtypeStruct((_B, 1000), jnp.float32),
    )(x, t, b)


def kernel(x, manyshotTemp, mediumshotTemp, fewshotTemp, manyshotBias,
           mediumshotBias, fewshotBias, many_mask, med_mask, few_mask):
    t = jnp.concatenate([manyshotTemp, mediumshotTemp, fewshotTemp], axis=1)
    b = jnp.concatenate([manyshotBias, mediumshotBias, fewshotBias], axis=1)
    return _run(x, t, b)
